# baseline (speedup 1.0000x reference)
import functools

import jax
import jax.numpy as jnp
from jax import lax
from jax.experimental import pallas as pl
from jax.experimental.pallas import tpu as pltpu
from jax.experimental.pallas import tpu_sc as plsc

SEQ = 200
DIM = 64
NC, NS = 2, 16
NW = NC * NS
CHUNK = 400
SUB = 80
NSUB = CHUNK // SUB


def _sc_body(idx_hbm, tok_hbm, posb_hbm, out_hbm, idx_v, rows_v, pos_v, gsem):
    per_w = idx_hbm.shape[0] // NW
    nchunk = per_w // CHUNK
    wid = lax.axis_index("s") * NC + lax.axis_index("c")
    base = wid * per_w

    pltpu.sync_copy(posb_hbm, pos_v)

    def chunk_body(ci, _):
        rbase = base + ci * CHUNK
        pltpu.sync_copy(idx_hbm.at[pl.ds(rbase, CHUNK)], idx_v)
        copies = [
            pltpu.async_copy(
                tok_hbm.at[idx_v.at[pl.ds(j * SUB, SUB)]],
                rows_v.at[pl.ds(j * SUB, SUB)],
                gsem,
            )
            for j in range(NSUB)
        ]
        for c in copies:
            c.wait()

        def fma_body(r, _):
            for u in range(DIM // 16):
                sl = pl.ds(u * 16, 16)
                rows_v[r, sl] = rows_v[r, sl] * 8.0 + pos_v[r, sl]
            return _

        lax.fori_loop(0, CHUNK, fma_body, 0, unroll=2)
        pltpu.sync_copy(rows_v, out_hbm.at[pl.ds(rbase, CHUNK)])
        return _

    lax.fori_loop(0, nchunk, chunk_body, 0)


def kernel(inputs, token_table, position_table):
    batch, seq = inputs.shape
    vocab, dim = token_table.shape
    flat = batch * seq
    idx_flat = inputs.reshape(flat).astype(jnp.int32)
    posb = jnp.tile(position_table, (CHUNK // SEQ, 1))

    mesh = plsc.VectorSubcoreMesh(
        core_axis_name="c", subcore_axis_name="s", num_cores=NC, num_subcores=NS
    )
    call = pl.kernel(
        _sc_body,
        out_type=jax.ShapeDtypeStruct((flat, dim), jnp.float32),
        mesh=mesh,
        scratch_types=[
            pltpu.VMEM((CHUNK,), jnp.int32),
            pltpu.VMEM((CHUNK, DIM), jnp.float32),
            pltpu.VMEM((CHUNK, DIM), jnp.float32),
            pltpu.SemaphoreType.DMA,
        ],
        compiler_params=pltpu.CompilerParams(use_tc_tiling_on_sc=False),
    )
    out_flat = call(idx_flat, token_table, posb)
    return out_flat.reshape(batch, seq, dim)


# speedup vs baseline: 1.3875x; 1.3875x over previous
import jax
import jax.numpy as jnp
from jax import lax
from jax.experimental import pallas as pl
from jax.experimental.pallas import tpu as pltpu
from jax.experimental.pallas import tpu_sc as plsc

SEQ = 200
DIM = 64
NC, NS = 2, 16
NW = NC * NS
CHUNK = 400
SUB = 80
NSUB = CHUNK // SUB


def _sc_body(idx_hbm, tok_hbm, posb_hbm, out_hbm,
             idx_v, rows0, rows1, pos_v, gsem0, gsem1, osem0, osem1):
    per_w = idx_hbm.shape[0] // NW
    nchunk = per_w // CHUNK
    wid = lax.axis_index("s") * NC + lax.axis_index("c")
    base = wid * per_w

    rows = (rows0, rows1)
    gsems = (gsem0, gsem1)
    osems = (osem0, osem1)

    pltpu.sync_copy(posb_hbm, pos_v)
    pltpu.sync_copy(idx_hbm.at[pl.ds(base, per_w)], idx_v)

    def fire_gather(ci, b):
        off = pl.multiple_of(ci * CHUNK, CHUNK)
        for j in range(NSUB):
            pltpu.async_copy(
                tok_hbm.at[idx_v.at[pl.ds(off + j * SUB, SUB)]],
                rows[b].at[pl.ds(j * SUB, SUB)],
                gsems[b],
            )

    def drain_gather(b):
        for j in range(NSUB):
            pltpu.make_async_copy(
                tok_hbm.at[pl.ds(0, SUB)],
                rows[b].at[pl.ds(j * SUB, SUB)],
                gsems[b],
            ).wait()

    def fire_out(ci, b):
        off = pl.multiple_of(base + ci * CHUNK, CHUNK)
        pltpu.async_copy(rows[b], out_hbm.at[pl.ds(off, CHUNK)], osems[b])

    def drain_out(b):
        pltpu.make_async_copy(rows[b], out_hbm.at[pl.ds(0, CHUNK)], osems[b]).wait()

    def fma(b):
        rb = rows[b]

        @plsc.parallel_loop(0, CHUNK, 1, unroll=4)
        def _(r):
            for u in range(DIM // 16):
                sl = pl.ds(u * 16, 16)
                rb[r, sl] = rb[r, sl] * 8.0 + pos_v[r, sl]

    fire_gather(0, 0)
    fire_gather(1, 1)
    drain_gather(0)
    fma(0)
    fire_out(0, 0)

    def pair(t, _):
        ci = 2 * t + 1
        drain_out(0)
        fire_gather(ci + 1, 0)
        drain_gather(1)
        fma(1)
        fire_out(ci, 1)

        drain_out(1)
        fire_gather(ci + 2, 1)
        drain_gather(0)
        fma(0)
        fire_out(ci + 1, 0)
        return _

    lax.fori_loop(0, (nchunk - 2) // 2, pair, 0)

    drain_gather(1)
    fma(1)
    fire_out(nchunk - 1, 1)
    drain_out(0)
    drain_out(1)


def kernel(inputs, token_table, position_table):
    batch, seq = inputs.shape
    vocab, dim = token_table.shape
    flat = batch * seq
    idx_flat = inputs.reshape(flat).astype(jnp.int32)
    posb = jnp.tile(position_table, (CHUNK // SEQ, 1))

    mesh = plsc.VectorSubcoreMesh(
        core_axis_name="c", subcore_axis_name="s", num_cores=NC, num_subcores=NS
    )
    per_w = flat // NW
    call = pl.kernel(
        _sc_body,
        out_type=jax.ShapeDtypeStruct((flat, dim), jnp.float32),
        mesh=mesh,
        scratch_types=[
            pltpu.VMEM((per_w,), jnp.int32),
            pltpu.VMEM((CHUNK, DIM), jnp.float32),
            pltpu.VMEM((CHUNK, DIM), jnp.float32),
            pltpu.VMEM((CHUNK, DIM), jnp.float32),
            pltpu.SemaphoreType.DMA,
            pltpu.SemaphoreType.DMA,
            pltpu.SemaphoreType.DMA,
            pltpu.SemaphoreType.DMA,
        ],
        compiler_params=pltpu.CompilerParams(use_tc_tiling_on_sc=False),
    )
    out_flat = call(idx_flat, token_table, posb)
    return out_flat.reshape(batch, seq, dim)


# speedup vs baseline: 1.4098x; 1.0161x over previous
import jax
import jax.numpy as jnp
from jax import lax
from jax.experimental import pallas as pl
from jax.experimental.pallas import tpu as pltpu
from jax.experimental.pallas import tpu_sc as plsc

NC, NS = 2, 16
NW = NC * NS
BB = 128
DIM = 64


def _sc_body(idxT_hbm, tok_hbm, pos_hbm, out_hbm,
             idx_v, rows0, rows1, tb0, tb1, pos_v,
             gsem0, gsem1, osem0, osem1):
    seq = idxT_hbm.shape[0]
    wid = lax.axis_index("s") * NC + lax.axis_index("c")
    b0 = wid * BB

    rows = (rows0, rows1)
    tbs = (tb0, tb1)
    gsems = (gsem0, gsem1)
    osems = (osem0, osem1)

    pltpu.sync_copy(pos_hbm, pos_v)
    pltpu.sync_copy(idxT_hbm.at[:, pl.ds(b0, BB)], idx_v)

    def fire_gather(s, b):
        pltpu.async_copy(tok_hbm.at[idx_v.at[s]], rows[b], gsems[b])

    def drain_gather(b):
        pltpu.make_async_copy(tok_hbm.at[pl.ds(0, BB)], rows[b], gsems[b]).wait()

    def fire_out(s, b):
        pltpu.async_copy(tbs[b], out_hbm.at[pl.ds(b0, BB), s], osems[b])

    def drain_out(b):
        pltpu.make_async_copy(tbs[b], out_hbm.at[pl.ds(0, BB), 0], osems[b]).wait()

    def compute(s, b):
        rb, tb = rows[b], tbs[b]
        p = [pos_v[s, pl.ds(16 * u, 16)] for u in range(DIM // 16)]

        @plsc.parallel_loop(0, BB, 1, unroll=4)
        def _(r):
            for u in range(DIM // 16):
                sl = pl.ds(16 * u, 16)
                tb[r, sl] = rb[r, sl] * 8.0 + p[u]

    fire_gather(0, 0)
    fire_gather(1, 1)
    drain_gather(0)
    compute(0, 0)
    fire_out(0, 0)
    fire_gather(2, 0)
    drain_gather(1)
    compute(1, 1)
    fire_out(1, 1)
    fire_gather(3, 1)

    def pair(t, _):
        s = 2 * t + 2
        drain_out(0)
        drain_gather(0)
        compute(s, 0)
        fire_out(s, 0)
        fire_gather(s + 2, 0)

        drain_out(1)
        drain_gather(1)
        compute(s + 1, 1)
        fire_out(s + 1, 1)
        fire_gather(s + 3, 1)
        return _

    lax.fori_loop(0, (seq - 4) // 2, pair, 0)

    for s, b in ((seq - 2, 0), (seq - 1, 1)):
        drain_out(b)
        drain_gather(b)
        compute(s, b)
        fire_out(s, b)
    drain_out(0)
    drain_out(1)


def kernel(inputs, token_table, position_table):
    batch, seq = inputs.shape
    vocab, dim = token_table.shape
    idxT = inputs.T.astype(jnp.int32)

    mesh = plsc.VectorSubcoreMesh(
        core_axis_name="c", subcore_axis_name="s", num_cores=NC, num_subcores=NS
    )
    call = pl.kernel(
        _sc_body,
        out_type=jax.ShapeDtypeStruct((batch, seq, dim), jnp.float32),
        mesh=mesh,
        scratch_types=[
            pltpu.VMEM((seq, BB), jnp.int32),
            pltpu.VMEM((BB, dim), jnp.float32),
            pltpu.VMEM((BB, dim), jnp.float32),
            pltpu.VMEM((BB, dim), jnp.float32),
            pltpu.VMEM((BB, dim), jnp.float32),
            pltpu.VMEM((seq, dim), jnp.float32),
            pltpu.SemaphoreType.DMA,
            pltpu.SemaphoreType.DMA,
            pltpu.SemaphoreType.DMA,
            pltpu.SemaphoreType.DMA,
        ],
        compiler_params=pltpu.CompilerParams(use_tc_tiling_on_sc=False),
    )
    return call(idxT, token_table, position_table)
